# baseline (device time: 246324 ns/iter reference)
import jax
import jax.numpy as jnp
from jax import lax
from jax.experimental import pallas as pl
from jax.experimental.pallas import tpu as pltpu

N_DEV = 4
SCALE = 0.08838834764831843


def _ring_allreduce(p):
    m, n = p.shape

    def body(p_ref, out_ref, comm_ref, send_sems, recv_sems):
        my = lax.axis_index("i")
        left = lax.rem(my + (N_DEV - 1), N_DEV)
        right = lax.rem(my + 1, N_DEV)

        barrier_sem = pltpu.get_barrier_semaphore()
        for nbr in (left, right):
            pl.semaphore_signal(
                barrier_sem, inc=1,
                device_id=(nbr,), device_id_type=pl.DeviceIdType.MESH,
            )
        pl.semaphore_wait(barrier_sem, 2)

        out_ref[...] = p_ref[...]
        comm_ref[0] = p_ref[...]

        for h in range(N_DEV - 1):
            rdma = pltpu.make_async_remote_copy(
                src_ref=comm_ref.at[h],
                dst_ref=comm_ref.at[h + 1],
                send_sem=send_sems.at[h],
                recv_sem=recv_sems.at[h],
                device_id=(right,),
                device_id_type=pl.DeviceIdType.MESH,
            )
            rdma.start()
            rdma.wait()
            out_ref[...] += comm_ref[h + 1]

    return pl.pallas_call(
        body,
        out_shape=jax.ShapeDtypeStruct((m, n), jnp.float32),
        in_specs=[pl.BlockSpec(memory_space=pltpu.VMEM)],
        out_specs=pl.BlockSpec(memory_space=pltpu.VMEM),
        scratch_shapes=[
            pltpu.VMEM((N_DEV, m, n), jnp.float32),
            pltpu.SemaphoreType.DMA((N_DEV - 1,)),
            pltpu.SemaphoreType.DMA((N_DEV - 1,)),
        ],
        compiler_params=pltpu.CompilerParams(collective_id=0),
    )(p)


def kernel(x, Wq, Wo, K_ext, V_ext):
    b, sq, d = x.shape
    dh = 128
    h_loc = Wq.shape[1] // dh
    i = lax.axis_index("i")

    xb = x.astype(jnp.bfloat16)
    q = jnp.einsum(
        "bsd,df->bsf", xb, Wq.astype(jnp.bfloat16),
        preferred_element_type=jnp.float32,
    ).reshape(b, sq, h_loc, dh).astype(jnp.bfloat16)

    k = lax.dynamic_slice_in_dim(K_ext, i * h_loc, h_loc, axis=2)
    v = lax.dynamic_slice_in_dim(V_ext, i * h_loc, h_loc, axis=2)

    s = jnp.einsum(
        "bihd,bjhd->bhij", q, k.astype(jnp.bfloat16),
        preferred_element_type=jnp.float32,
    ) * SCALE
    p = jax.nn.softmax(s, axis=-1)
    o = jnp.einsum(
        "bhij,bjhd->bihd", p.astype(jnp.bfloat16), v.astype(jnp.bfloat16),
        preferred_element_type=jnp.float32,
    ).reshape(b, sq, h_loc * dh)

    partial = jnp.einsum(
        "bsf,fd->bsd", o.astype(jnp.bfloat16), Wo.astype(jnp.bfloat16),
        preferred_element_type=jnp.float32,
    )

    out = _ring_allreduce(partial.reshape(b * sq, d))
    return out.reshape(b, sq, d)


# device time: 150817 ns/iter; 1.6333x vs baseline; 1.6333x over previous
import jax
import jax.numpy as jnp
from jax import lax
from jax.experimental import pallas as pl
from jax.experimental.pallas import tpu as pltpu

N_DEV = 4
SCALE = 0.08838834764831843


def _ring_allreduce(p):
    m, n = p.shape
    mc = m // N_DEV

    def body(p_ref, out_ref, comm_ref, send_sems, recv_sems):
        my = lax.axis_index("i")
        left = lax.rem(my + (N_DEV - 1), N_DEV)
        right = lax.rem(my + 1, N_DEV)

        def chunk(ref, idx):
            start = lax.rem(idx + 2 * N_DEV, N_DEV) * mc
            return ref[pl.ds(start, mc), :]

        barrier_sem = pltpu.get_barrier_semaphore()
        for nbr in (left, right):
            pl.semaphore_signal(
                barrier_sem, inc=1,
                device_id=(nbr,), device_id_type=pl.DeviceIdType.MESH,
            )
        pl.semaphore_wait(barrier_sem, 2)

        comm_ref[0] = chunk(p_ref, my)
        for s in range(N_DEV - 1):
            rdma = pltpu.make_async_remote_copy(
                src_ref=comm_ref.at[s],
                dst_ref=comm_ref.at[s + 1],
                send_sem=send_sems.at[s],
                recv_sem=recv_sems.at[s],
                device_id=(right,),
                device_id_type=pl.DeviceIdType.MESH,
            )
            rdma.start()
            rdma.wait()
            acc = comm_ref[s + 1].astype(jnp.float32) + chunk(
                p_ref, my - s - 1
            ).astype(jnp.float32)
            comm_ref[s + 1] = acc.astype(p_ref.dtype)

        g = lax.rem(my + 1, N_DEV)
        out_ref[pl.ds(g * mc, mc), :] = comm_ref[N_DEV - 1]

        for t in range(N_DEV - 1):
            rdma = pltpu.make_async_remote_copy(
                src_ref=comm_ref.at[N_DEV - 1 + t],
                dst_ref=comm_ref.at[N_DEV + t],
                send_sem=send_sems.at[N_DEV - 1 + t],
                recv_sem=recv_sems.at[N_DEV - 1 + t],
                device_id=(right,),
                device_id_type=pl.DeviceIdType.MESH,
            )
            rdma.start()
            rdma.wait()
            start = lax.rem(my - t + 2 * N_DEV, N_DEV) * mc
            out_ref[pl.ds(start, mc), :] = comm_ref[N_DEV + t]

    return pl.pallas_call(
        body,
        out_shape=jax.ShapeDtypeStruct((m, n), p.dtype),
        in_specs=[pl.BlockSpec(memory_space=pltpu.VMEM)],
        out_specs=pl.BlockSpec(memory_space=pltpu.VMEM),
        scratch_shapes=[
            pltpu.VMEM((2 * N_DEV - 1, mc, n), p.dtype),
            pltpu.SemaphoreType.DMA((2 * (N_DEV - 1),)),
            pltpu.SemaphoreType.DMA((2 * (N_DEV - 1),)),
        ],
        compiler_params=pltpu.CompilerParams(collective_id=0),
    )(p)


def kernel(x, Wq, Wo, K_ext, V_ext):
    b, sq, d = x.shape
    dh = 128
    h_loc = Wq.shape[1] // dh
    i = lax.axis_index("i")

    xb = x.astype(jnp.bfloat16)
    q = jnp.einsum(
        "bsd,df->bsf", xb, Wq.astype(jnp.bfloat16),
        preferred_element_type=jnp.float32,
    ).reshape(b, sq, h_loc, dh).astype(jnp.bfloat16)

    k = lax.dynamic_slice_in_dim(K_ext, i * h_loc, h_loc, axis=2)
    v = lax.dynamic_slice_in_dim(V_ext, i * h_loc, h_loc, axis=2)

    s = jnp.einsum(
        "bihd,bjhd->bhij", q, k.astype(jnp.bfloat16),
        preferred_element_type=jnp.float32,
    ) * SCALE
    p = jax.nn.softmax(s, axis=-1)
    o = jnp.einsum(
        "bhij,bjhd->bihd", p.astype(jnp.bfloat16), v.astype(jnp.bfloat16),
        preferred_element_type=jnp.float32,
    ).reshape(b, sq, h_loc * dh)

    partial = jnp.einsum(
        "bsf,fd->bsd", o.astype(jnp.bfloat16), Wo.astype(jnp.bfloat16),
        preferred_element_type=jnp.float32,
    )

    out = _ring_allreduce(partial.reshape(b * sq, d).astype(jnp.bfloat16))
    return out.reshape(b, sq, d).astype(jnp.float32)


# device time: 116424 ns/iter; 2.1157x vs baseline; 1.2954x over previous
import jax
import jax.numpy as jnp
from jax import lax
from jax.experimental import pallas as pl
from jax.experimental.pallas import tpu as pltpu

N_DEV = 4
SCALE = 0.08838834764831843


def _fused(x2, wq, wo, kt, vt):
    m, d = x2.shape
    b_sz, h_loc, dh, skv = kt.shape
    mc = m // N_DEV

    def body(x_ref, wq_ref, wo_ref, kt_ref, vt_ref, out_ref,
             q_ref, o_ref, part_ref, comm_ref, send_sems, recv_sems):
        my = lax.axis_index("i")
        left = lax.rem(my + (N_DEV - 1), N_DEV)
        right = lax.rem(my + 1, N_DEV)

        def chunk(ref, idx):
            start = lax.rem(idx + 2 * N_DEV, N_DEV) * mc
            return ref[pl.ds(start, mc), :]

        barrier_sem = pltpu.get_barrier_semaphore()
        for nbr in (left, right):
            pl.semaphore_signal(
                barrier_sem, inc=1,
                device_id=(nbr,), device_id_type=pl.DeviceIdType.MESH,
            )
        pl.semaphore_wait(barrier_sem, 2)

        q_ref[...] = jnp.dot(
            x_ref[...], wq_ref[...], preferred_element_type=jnp.float32
        ).astype(jnp.bfloat16)

        for bb in range(b_sz):
            for h in range(h_loc):
                qbh = q_ref[bb * mc:(bb + 1) * mc, h * dh:(h + 1) * dh]
                s = jnp.dot(
                    qbh, kt_ref[bb, h], preferred_element_type=jnp.float32
                ) * SCALE
                e = jnp.exp(s - jnp.max(s, axis=1, keepdims=True))
                p = (e / jnp.sum(e, axis=1, keepdims=True)).astype(jnp.bfloat16)
                o_ref[:, h * dh:(h + 1) * dh] = jnp.dot(
                    p, vt_ref[bb, h], preferred_element_type=jnp.float32
                ).astype(jnp.bfloat16)
            part_ref[bb * mc:(bb + 1) * mc, :] = jnp.dot(
                o_ref[...], wo_ref[...], preferred_element_type=jnp.float32
            ).astype(jnp.bfloat16)

        comm_ref[0] = chunk(part_ref, my)
        for s in range(N_DEV - 1):
            rdma = pltpu.make_async_remote_copy(
                src_ref=comm_ref.at[s],
                dst_ref=comm_ref.at[s + 1],
                send_sem=send_sems.at[s],
                recv_sem=recv_sems.at[s],
                device_id=(right,),
                device_id_type=pl.DeviceIdType.MESH,
            )
            rdma.start()
            rdma.wait()
            acc = comm_ref[s + 1].astype(jnp.float32) + chunk(
                part_ref, my - s - 1
            ).astype(jnp.float32)
            comm_ref[s + 1] = acc.astype(jnp.bfloat16)

        g = lax.rem(my + 1, N_DEV)
        out_ref[pl.ds(g * mc, mc), :] = comm_ref[N_DEV - 1].astype(jnp.float32)

        for t in range(N_DEV - 1):
            rdma = pltpu.make_async_remote_copy(
                src_ref=comm_ref.at[N_DEV - 1 + t],
                dst_ref=comm_ref.at[N_DEV + t],
                send_sem=send_sems.at[N_DEV - 1 + t],
                recv_sem=recv_sems.at[N_DEV - 1 + t],
                device_id=(right,),
                device_id_type=pl.DeviceIdType.MESH,
            )
            rdma.start()
            rdma.wait()
            start = lax.rem(my - t + 2 * N_DEV, N_DEV) * mc
            out_ref[pl.ds(start, mc), :] = comm_ref[N_DEV + t].astype(
                jnp.float32
            )

    return pl.pallas_call(
        body,
        out_shape=jax.ShapeDtypeStruct((m, d), jnp.float32),
        in_specs=[pl.BlockSpec(memory_space=pltpu.VMEM)] * 5,
        out_specs=pl.BlockSpec(memory_space=pltpu.VMEM),
        scratch_shapes=[
            pltpu.VMEM((m, h_loc * dh), jnp.bfloat16),
            pltpu.VMEM((mc, h_loc * dh), jnp.bfloat16),
            pltpu.VMEM((m, d), jnp.bfloat16),
            pltpu.VMEM((2 * N_DEV - 1, mc, d), jnp.bfloat16),
            pltpu.SemaphoreType.DMA((2 * (N_DEV - 1),)),
            pltpu.SemaphoreType.DMA((2 * (N_DEV - 1),)),
        ],
        compiler_params=pltpu.CompilerParams(collective_id=0),
    )(x2, wq, wo, kt, vt)


def kernel(x, Wq, Wo, K_ext, V_ext):
    b, sq, d = x.shape
    dh = 128
    h_loc = Wq.shape[1] // dh
    i = lax.axis_index("i")

    x2 = x.reshape(b * sq, d).astype(jnp.bfloat16)
    wq = Wq.astype(jnp.bfloat16)
    wo = Wo.astype(jnp.bfloat16)
    k = lax.dynamic_slice_in_dim(K_ext, i * h_loc, h_loc, axis=2)
    v = lax.dynamic_slice_in_dim(V_ext, i * h_loc, h_loc, axis=2)
    kt = jnp.transpose(k.astype(jnp.bfloat16), (0, 2, 3, 1))
    vt = jnp.transpose(v.astype(jnp.bfloat16), (0, 2, 1, 3))

    out = _fused(x2, wq, wo, kt, vt)
    return out.reshape(b, sq, d)


# device time: 98202 ns/iter; 2.5083x vs baseline; 1.1856x over previous
import jax
import jax.numpy as jnp
from jax import lax
from jax.experimental import pallas as pl
from jax.experimental.pallas import tpu as pltpu

N_DEV = 4
SCALE = 0.08838834764831843


def _fused(x2, wq, wo, kt, vt):
    m, d = x2.shape
    b_sz, h_loc, dh, skv = kt.shape
    mc = m // N_DEV

    def body(x_ref, wq_ref, wo_ref, kt_ref, vt_ref, out_ref,
             o_ref, part_ref, comm_ref, send_sems, recv_sems):
        my = lax.axis_index("i")
        left = lax.rem(my + (N_DEV - 1), N_DEV)
        right = lax.rem(my + 1, N_DEV)

        barrier_sem = pltpu.get_barrier_semaphore()
        for nbr in (left, right):
            pl.semaphore_signal(
                barrier_sem, inc=1,
                device_id=(nbr,), device_id_type=pl.DeviceIdType.MESH,
            )
        pl.semaphore_wait(barrier_sem, 2)

        def compute_batch(bidx, dst):
            qb = jnp.dot(
                x_ref[pl.ds(bidx * mc, mc), :], wq_ref[...],
                preferred_element_type=jnp.float32,
            ).astype(jnp.bfloat16)
            for h in range(h_loc):
                s = jnp.dot(
                    qb[:, h * dh:(h + 1) * dh], kt_ref[bidx, h],
                    preferred_element_type=jnp.float32,
                ) * SCALE
                e = jnp.exp(s - jnp.max(s, axis=1, keepdims=True))
                p = (e / jnp.sum(e, axis=1, keepdims=True)).astype(jnp.bfloat16)
                o_ref[:, h * dh:(h + 1) * dh] = jnp.dot(
                    p, vt_ref[bidx, h], preferred_element_type=jnp.float32
                ).astype(jnp.bfloat16)
            dst[...] = jnp.dot(
                o_ref[...], wo_ref[...], preferred_element_type=jnp.float32
            ).astype(jnp.bfloat16)

        compute_batch(my, comm_ref.at[0])
        for s in range(N_DEV - 1):
            rdma = pltpu.make_async_remote_copy(
                src_ref=comm_ref.at[s],
                dst_ref=comm_ref.at[s + 1],
                send_sem=send_sems.at[s],
                recv_sem=recv_sems.at[s],
                device_id=(right,),
                device_id_type=pl.DeviceIdType.MESH,
            )
            rdma.start()
            compute_batch(
                lax.rem(my - s - 1 + 2 * N_DEV, N_DEV), part_ref.at[s]
            )
            rdma.wait()
            acc = comm_ref[s + 1].astype(jnp.float32) + part_ref[s].astype(
                jnp.float32
            )
            comm_ref[s + 1] = acc.astype(jnp.bfloat16)

        g = lax.rem(my + 1, N_DEV)
        out_ref[pl.ds(g * mc, mc), :] = comm_ref[N_DEV - 1].astype(jnp.float32)

        for t in range(N_DEV - 1):
            rdma = pltpu.make_async_remote_copy(
                src_ref=comm_ref.at[N_DEV - 1 + t],
                dst_ref=comm_ref.at[N_DEV + t],
                send_sem=send_sems.at[N_DEV - 1 + t],
                recv_sem=recv_sems.at[N_DEV - 1 + t],
                device_id=(right,),
                device_id_type=pl.DeviceIdType.MESH,
            )
            rdma.start()
            rdma.wait()
            start = lax.rem(my - t + 2 * N_DEV, N_DEV) * mc
            out_ref[pl.ds(start, mc), :] = comm_ref[N_DEV + t].astype(
                jnp.float32
            )

    return pl.pallas_call(
        body,
        out_shape=jax.ShapeDtypeStruct((m, d), jnp.float32),
        in_specs=[pl.BlockSpec(memory_space=pltpu.VMEM)] * 5,
        out_specs=pl.BlockSpec(memory_space=pltpu.VMEM),
        scratch_shapes=[
            pltpu.VMEM((mc, h_loc * dh), jnp.bfloat16),
            pltpu.VMEM((N_DEV - 1, mc, d), jnp.bfloat16),
            pltpu.VMEM((2 * N_DEV - 1, mc, d), jnp.bfloat16),
            pltpu.SemaphoreType.DMA((2 * (N_DEV - 1),)),
            pltpu.SemaphoreType.DMA((2 * (N_DEV - 1),)),
        ],
        compiler_params=pltpu.CompilerParams(collective_id=0),
    )(x2, wq, wo, kt, vt)


def kernel(x, Wq, Wo, K_ext, V_ext):
    b, sq, d = x.shape
    dh = 128
    h_loc = Wq.shape[1] // dh
    i = lax.axis_index("i")

    x2 = x.reshape(b * sq, d).astype(jnp.bfloat16)
    wq = Wq.astype(jnp.bfloat16)
    wo = Wo.astype(jnp.bfloat16)
    k = lax.dynamic_slice_in_dim(K_ext, i * h_loc, h_loc, axis=2)
    v = lax.dynamic_slice_in_dim(V_ext, i * h_loc, h_loc, axis=2)
    kt = jnp.transpose(k.astype(jnp.bfloat16), (0, 2, 3, 1))
    vt = jnp.transpose(v.astype(jnp.bfloat16), (0, 2, 1, 3))

    out = _fused(x2, wq, wo, kt, vt)
    return out.reshape(b, sq, d)


# device time: 92851 ns/iter; 2.6529x vs baseline; 1.0576x over previous
import jax
import jax.numpy as jnp
from jax import lax
from jax.experimental import pallas as pl
from jax.experimental.pallas import tpu as pltpu

N_DEV = 4
SCALE = 0.08838834764831843


def _fused(x2, wq, wo, kt, vt):
    m, d = x2.shape
    b_sz, h_loc, skv, dh = kt.shape
    mc = m // N_DEV

    def body(x_ref, wq_ref, wo_ref, kt_ref, vt_ref, out_ref,
             o_ref, part_ref, comm_ref, send_sems, recv_sems):
        my = lax.axis_index("i")
        left = lax.rem(my + (N_DEV - 1), N_DEV)
        right = lax.rem(my + 1, N_DEV)

        barrier_sem = pltpu.get_barrier_semaphore()
        for nbr in (left, right):
            pl.semaphore_signal(
                barrier_sem, inc=1,
                device_id=(nbr,), device_id_type=pl.DeviceIdType.MESH,
            )
        pl.semaphore_wait(barrier_sem, 2)

        def compute_batch(bidx, dst):
            qb = jnp.dot(
                x_ref[pl.ds(bidx * mc, mc), :], wq_ref[...],
                preferred_element_type=jnp.float32,
            ).astype(jnp.bfloat16)
            for h in range(h_loc):
                s = lax.dot_general(
                    qb[:, h * dh:(h + 1) * dh], kt_ref[bidx, h],
                    (((1,), (1,)), ((), ())),
                    preferred_element_type=jnp.float32,
                ) * SCALE
                e = jnp.exp(s - jnp.max(s, axis=1, keepdims=True))
                p = (e / jnp.sum(e, axis=1, keepdims=True)).astype(jnp.bfloat16)
                o_ref[:, h * dh:(h + 1) * dh] = jnp.dot(
                    p, vt_ref[bidx, h], preferred_element_type=jnp.float32
                ).astype(jnp.bfloat16)
            dst[...] = jnp.dot(
                o_ref[...], wo_ref[...], preferred_element_type=jnp.float32
            ).astype(jnp.bfloat16)

        compute_batch(my, comm_ref.at[0])
        for s in range(N_DEV - 1):
            rdma = pltpu.make_async_remote_copy(
                src_ref=comm_ref.at[s],
                dst_ref=comm_ref.at[s + 1],
                send_sem=send_sems.at[s],
                recv_sem=recv_sems.at[s],
                device_id=(right,),
                device_id_type=pl.DeviceIdType.MESH,
            )
            rdma.start()
            compute_batch(
                lax.rem(my - s - 1 + 2 * N_DEV, N_DEV), part_ref.at[s]
            )
            rdma.wait()
            acc = comm_ref[s + 1].astype(jnp.float32) + part_ref[s].astype(
                jnp.float32
            )
            comm_ref[s + 1] = acc.astype(jnp.bfloat16)

        g = lax.rem(my + 1, N_DEV)
        out_ref[pl.ds(g * mc, mc), :] = comm_ref[N_DEV - 1].astype(jnp.float32)

        for t in range(N_DEV - 1):
            rdma = pltpu.make_async_remote_copy(
                src_ref=comm_ref.at[N_DEV - 1 + t],
                dst_ref=comm_ref.at[N_DEV + t],
                send_sem=send_sems.at[N_DEV - 1 + t],
                recv_sem=recv_sems.at[N_DEV - 1 + t],
                device_id=(right,),
                device_id_type=pl.DeviceIdType.MESH,
            )
            rdma.start()
            rdma.wait()
            start = lax.rem(my - t + 2 * N_DEV, N_DEV) * mc
            out_ref[pl.ds(start, mc), :] = comm_ref[N_DEV + t].astype(
                jnp.float32
            )

    return pl.pallas_call(
        body,
        out_shape=jax.ShapeDtypeStruct((m, d), jnp.float32),
        in_specs=[pl.BlockSpec(memory_space=pltpu.VMEM)] * 5,
        out_specs=pl.BlockSpec(memory_space=pltpu.VMEM),
        scratch_shapes=[
            pltpu.VMEM((mc, h_loc * dh), jnp.bfloat16),
            pltpu.VMEM((N_DEV - 1, mc, d), jnp.bfloat16),
            pltpu.VMEM((2 * N_DEV - 1, mc, d), jnp.bfloat16),
            pltpu.SemaphoreType.DMA((2 * (N_DEV - 1),)),
            pltpu.SemaphoreType.DMA((2 * (N_DEV - 1),)),
        ],
        compiler_params=pltpu.CompilerParams(collective_id=0),
    )(x2, wq, wo, kt, vt)


def kernel(x, Wq, Wo, K_ext, V_ext):
    b, sq, d = x.shape
    dh = 128
    h_loc = Wq.shape[1] // dh
    i = lax.axis_index("i")

    x2 = x.reshape(b * sq, d).astype(jnp.bfloat16)
    wq = Wq.astype(jnp.bfloat16)
    wo = Wo.astype(jnp.bfloat16)
    k = lax.dynamic_slice_in_dim(K_ext, i * h_loc, h_loc, axis=2)
    v = lax.dynamic_slice_in_dim(V_ext, i * h_loc, h_loc, axis=2)
    kt = jnp.transpose(k.astype(jnp.bfloat16), (0, 2, 1, 3))
    vt = jnp.transpose(v.astype(jnp.bfloat16), (0, 2, 1, 3))

    out = _fused(x2, wq, wo, kt, vt)
    return out.reshape(b, sq, d)


# device time: 90063 ns/iter; 2.7350x vs baseline; 1.0310x over previous
import jax
import jax.numpy as jnp
from jax import lax
from jax.experimental import pallas as pl
from jax.experimental.pallas import tpu as pltpu

N_DEV = 4
SCALE = 0.08838834764831843


def _fused(x2, wq, wo, kt, vt):
    m, d = x2.shape
    b_sz, h_loc, skv, dh = kt.shape
    mc = m // N_DEV

    def body(x_ref, wq_ref, wo_ref, kt_ref, vt_ref, out_ref,
             o_ref, part_ref, comm_ref, send_sems, recv_sems):
        my = lax.axis_index("i")
        left = lax.rem(my + (N_DEV - 1), N_DEV)
        right = lax.rem(my + 1, N_DEV)

        barrier_sem = pltpu.get_barrier_semaphore()
        for nbr in (left, right):
            pl.semaphore_signal(
                barrier_sem, inc=1,
                device_id=(nbr,), device_id_type=pl.DeviceIdType.MESH,
            )
        pl.semaphore_wait(barrier_sem, 2)

        def compute_batch(bidx, dst):
            qb = jnp.dot(
                x_ref[pl.ds(bidx * mc, mc), :], wq_ref[...],
                preferred_element_type=jnp.float32,
            ).astype(jnp.bfloat16)
            for h in range(h_loc):
                s = lax.dot_general(
                    qb[:, h * dh:(h + 1) * dh], kt_ref[bidx, h],
                    (((1,), (1,)), ((), ())),
                    preferred_element_type=jnp.float32,
                )
                e = jnp.exp(s)
                o = jnp.dot(
                    e.astype(jnp.bfloat16), vt_ref[bidx, h],
                    preferred_element_type=jnp.float32,
                ) / jnp.sum(e, axis=1, keepdims=True)
                o_ref[:, h * dh:(h + 1) * dh] = o.astype(jnp.bfloat16)
            dst[...] = jnp.dot(
                o_ref[...], wo_ref[...], preferred_element_type=jnp.float32
            ).astype(jnp.bfloat16)

        compute_batch(my, comm_ref.at[0])
        for s in range(N_DEV - 1):
            rdma = pltpu.make_async_remote_copy(
                src_ref=comm_ref.at[s],
                dst_ref=comm_ref.at[s + 1],
                send_sem=send_sems.at[s],
                recv_sem=recv_sems.at[s],
                device_id=(right,),
                device_id_type=pl.DeviceIdType.MESH,
            )
            rdma.start()
            compute_batch(
                lax.rem(my - s - 1 + 2 * N_DEV, N_DEV), part_ref.at[s]
            )
            rdma.wait()
            acc = comm_ref[s + 1].astype(jnp.float32) + part_ref[s].astype(
                jnp.float32
            )
            comm_ref[s + 1] = acc.astype(jnp.bfloat16)

        g = lax.rem(my + 1, N_DEV)
        out_ref[pl.ds(g * mc, mc), :] = comm_ref[N_DEV - 1].astype(jnp.float32)

        for t in range(N_DEV - 1):
            rdma = pltpu.make_async_remote_copy(
                src_ref=comm_ref.at[N_DEV - 1 + t],
                dst_ref=comm_ref.at[N_DEV + t],
                send_sem=send_sems.at[N_DEV - 1 + t],
                recv_sem=recv_sems.at[N_DEV - 1 + t],
                device_id=(right,),
                device_id_type=pl.DeviceIdType.MESH,
            )
            rdma.start()
            rdma.wait()
            start = lax.rem(my - t + 2 * N_DEV, N_DEV) * mc
            out_ref[pl.ds(start, mc), :] = comm_ref[N_DEV + t].astype(
                jnp.float32
            )

    return pl.pallas_call(
        body,
        out_shape=jax.ShapeDtypeStruct((m, d), jnp.float32),
        in_specs=[pl.BlockSpec(memory_space=pltpu.VMEM)] * 5,
        out_specs=pl.BlockSpec(memory_space=pltpu.VMEM),
        scratch_shapes=[
            pltpu.VMEM((mc, h_loc * dh), jnp.bfloat16),
            pltpu.VMEM((N_DEV - 1, mc, d), jnp.bfloat16),
            pltpu.VMEM((2 * N_DEV - 1, mc, d), jnp.bfloat16),
            pltpu.SemaphoreType.DMA((2 * (N_DEV - 1),)),
            pltpu.SemaphoreType.DMA((2 * (N_DEV - 1),)),
        ],
        compiler_params=pltpu.CompilerParams(collective_id=0),
    )(x2, wq, wo, kt, vt)


def kernel(x, Wq, Wo, K_ext, V_ext):
    b, sq, d = x.shape
    dh = 128
    h_loc = Wq.shape[1] // dh
    i = lax.axis_index("i")

    x2 = x.reshape(b * sq, d).astype(jnp.bfloat16)
    wq = (Wq * SCALE).astype(jnp.bfloat16)
    wo = Wo.astype(jnp.bfloat16)
    k = lax.dynamic_slice_in_dim(K_ext, i * h_loc, h_loc, axis=2)
    v = lax.dynamic_slice_in_dim(V_ext, i * h_loc, h_loc, axis=2)
    kt = jnp.transpose(k.astype(jnp.bfloat16), (0, 2, 1, 3))
    vt = jnp.transpose(v.astype(jnp.bfloat16), (0, 2, 1, 3))

    out = _fused(x2, wq, wo, kt, vt)
    return out.reshape(b, sq, d)


# device time: 81681 ns/iter; 3.0157x vs baseline; 1.1026x over previous
import jax
import jax.numpy as jnp
from jax import lax
from jax.experimental import pallas as pl
from jax.experimental.pallas import tpu as pltpu

N_DEV = 4
SCALE = 0.08838834764831843


def _fused(x2, wq, wo, kt, vt):
    m, d = x2.shape
    b_sz, h_loc, skv, dh = kt.shape
    mc = m // N_DEV

    def body(x_ref, wq_ref, wo_ref, kt_ref, vt_ref, out_ref,
             o_ref, part_ref, comm_ref, send_sems, recv_sems,
             ag_send_sems, ag_recv_sems):
        my = lax.axis_index("i")
        left = lax.rem(my + (N_DEV - 1), N_DEV)
        right = lax.rem(my + 1, N_DEV)

        barrier_sem = pltpu.get_barrier_semaphore()
        for nbr in (left, right):
            pl.semaphore_signal(
                barrier_sem, inc=1,
                device_id=(nbr,), device_id_type=pl.DeviceIdType.MESH,
            )
        pl.semaphore_wait(barrier_sem, 2)

        def compute_batch(bidx, dst):
            qb = jnp.dot(
                x_ref[pl.ds(bidx * mc, mc), :], wq_ref[...],
                preferred_element_type=jnp.float32,
            ).astype(jnp.bfloat16)
            for h in range(h_loc):
                s = lax.dot_general(
                    qb[:, h * dh:(h + 1) * dh], kt_ref[bidx, h],
                    (((1,), (1,)), ((), ())),
                    preferred_element_type=jnp.float32,
                )
                e = jnp.exp(s)
                o = jnp.dot(
                    e.astype(jnp.bfloat16), vt_ref[bidx, h],
                    preferred_element_type=jnp.float32,
                ) / jnp.sum(e, axis=1, keepdims=True)
                o_ref[:, h * dh:(h + 1) * dh] = o.astype(jnp.bfloat16)
            dst[...] = jnp.dot(
                o_ref[...], wo_ref[...], preferred_element_type=jnp.float32
            ).astype(jnp.bfloat16)

        compute_batch(my, comm_ref.at[0])
        for s in range(N_DEV - 1):
            rdma = pltpu.make_async_remote_copy(
                src_ref=comm_ref.at[s],
                dst_ref=comm_ref.at[s + 1],
                send_sem=send_sems.at[s],
                recv_sem=recv_sems.at[s],
                device_id=(right,),
                device_id_type=pl.DeviceIdType.MESH,
            )
            rdma.start()
            compute_batch(
                lax.rem(my - s - 1 + 2 * N_DEV, N_DEV), part_ref.at[s]
            )
            rdma.wait()
            acc = comm_ref[s + 1].astype(jnp.float32) + part_ref[s].astype(
                jnp.float32
            )
            comm_ref[s + 1] = acc.astype(jnp.bfloat16)

        g = lax.rem(my + 1, N_DEV)
        out_ref[pl.ds(g * mc, mc), :] = comm_ref[N_DEV - 1].astype(jnp.float32)

        ag_rdmas = []
        for j in range(1, N_DEV):
            tgt = lax.rem(my + j, N_DEV)
            m_slot = N_DEV - 1 - j
            rdma = pltpu.make_async_remote_copy(
                src_ref=comm_ref.at[N_DEV - 1],
                dst_ref=comm_ref.at[N_DEV + m_slot],
                send_sem=ag_send_sems.at[j - 1],
                recv_sem=ag_recv_sems.at[m_slot],
                device_id=(tgt,),
                device_id_type=pl.DeviceIdType.MESH,
            )
            rdma.start()
            ag_rdmas.append(rdma)

        for m_slot in (0, 2, 1):
            recv = pltpu.make_async_remote_copy(
                src_ref=comm_ref.at[N_DEV - 1],
                dst_ref=comm_ref.at[N_DEV + m_slot],
                send_sem=ag_send_sems.at[0],
                recv_sem=ag_recv_sems.at[m_slot],
                device_id=(my,),
                device_id_type=pl.DeviceIdType.MESH,
            )
            recv.wait_recv()
            origin = lax.rem(my + m_slot + 2, N_DEV)
            out_ref[pl.ds(origin * mc, mc), :] = comm_ref[
                N_DEV + m_slot
            ].astype(jnp.float32)

        for rdma in ag_rdmas:
            rdma.wait_send()

    return pl.pallas_call(
        body,
        out_shape=jax.ShapeDtypeStruct((m, d), jnp.float32),
        in_specs=[pl.BlockSpec(memory_space=pltpu.VMEM)] * 5,
        out_specs=pl.BlockSpec(memory_space=pltpu.VMEM),
        scratch_shapes=[
            pltpu.VMEM((mc, h_loc * dh), jnp.bfloat16),
            pltpu.VMEM((N_DEV - 1, mc, d), jnp.bfloat16),
            pltpu.VMEM((2 * N_DEV - 1, mc, d), jnp.bfloat16),
            pltpu.SemaphoreType.DMA((N_DEV - 1,)),
            pltpu.SemaphoreType.DMA((N_DEV - 1,)),
            pltpu.SemaphoreType.DMA((N_DEV - 1,)),
            pltpu.SemaphoreType.DMA((N_DEV - 1,)),
        ],
        compiler_params=pltpu.CompilerParams(collective_id=0),
    )(x2, wq, wo, kt, vt)


def kernel(x, Wq, Wo, K_ext, V_ext):
    b, sq, d = x.shape
    dh = 128
    h_loc = Wq.shape[1] // dh
    i = lax.axis_index("i")

    x2 = x.reshape(b * sq, d).astype(jnp.bfloat16)
    wq = (Wq * SCALE).astype(jnp.bfloat16)
    wo = Wo.astype(jnp.bfloat16)
    k = lax.dynamic_slice_in_dim(K_ext, i * h_loc, h_loc, axis=2)
    v = lax.dynamic_slice_in_dim(V_ext, i * h_loc, h_loc, axis=2)
    kt = jnp.transpose(k.astype(jnp.bfloat16), (0, 2, 1, 3))
    vt = jnp.transpose(v.astype(jnp.bfloat16), (0, 2, 1, 3))

    out = _fused(x2, wq, wo, kt, vt)
    return out.reshape(b, sq, d)


# device time: 52587 ns/iter; 4.6841x vs baseline; 1.5533x over previous
import jax
import jax.numpy as jnp
from jax import lax
from jax.experimental import pallas as pl
from jax.experimental.pallas import tpu as pltpu

N_DEV = 4
SCALE = 0.08838834764831843


def _fused(x2, wq, wo, k_ext, v_ext):
    m, d = x2.shape
    b_sz, skv, h_tot, dh = k_ext.shape
    h_loc = wq.shape[1] // dh
    mc = m // N_DEV

    def body(x_ref, wq_ref, wo_ref, khbm_ref, vhbm_ref, out_ref,
             o_ref, part_ref, comm_ref, kbuf, vbuf, load_sems,
             send_sems, recv_sems, ag_send_sems, ag_recv_sems):
        my = lax.axis_index("i")
        left = lax.rem(my + (N_DEV - 1), N_DEV)
        right = lax.rem(my + 1, N_DEV)
        base = my * h_loc

        def ridx(idx):
            return lax.rem(idx + 2 * N_DEV, N_DEV)

        def start_load(bidx, slot):
            copies = []
            for h in range(h_loc):
                copies.append(pltpu.make_async_copy(
                    khbm_ref.at[bidx, :, base + h, :],
                    kbuf.at[slot, h], load_sems.at[slot]))
                copies.append(pltpu.make_async_copy(
                    vhbm_ref.at[bidx, :, base + h, :],
                    vbuf.at[slot, h], load_sems.at[slot]))
            for c in copies:
                c.start()
            return copies

        loads = [start_load(my, 0), start_load(ridx(my - 1), 1)]

        barrier_sem = pltpu.get_barrier_semaphore()
        for nbr in (left, right):
            pl.semaphore_signal(
                barrier_sem, inc=1,
                device_id=(nbr,), device_id_type=pl.DeviceIdType.MESH,
            )
        pl.semaphore_wait(barrier_sem, 2)

        def compute_batch(bidx, slot, dst):
            qb = jnp.dot(
                x_ref[pl.ds(bidx * mc, mc), :], wq_ref[...],
                preferred_element_type=jnp.float32,
            ).astype(jnp.bfloat16)
            for h in range(h_loc):
                s = lax.dot_general(
                    qb[:, h * dh:(h + 1) * dh],
                    kbuf[slot, h].astype(jnp.bfloat16),
                    (((1,), (1,)), ((), ())),
                    preferred_element_type=jnp.float32,
                )
                e = jnp.exp(s)
                o = jnp.dot(
                    e.astype(jnp.bfloat16),
                    vbuf[slot, h].astype(jnp.bfloat16),
                    preferred_element_type=jnp.float32,
                ) / jnp.sum(e, axis=1, keepdims=True)
                o_ref[:, h * dh:(h + 1) * dh] = o.astype(jnp.bfloat16)
            dst[...] = jnp.dot(
                o_ref[...], wo_ref[...], preferred_element_type=jnp.float32
            ).astype(jnp.bfloat16)

        for c in loads[0]:
            c.wait()
        compute_batch(my, 0, comm_ref.at[0])
        for s in range(N_DEV - 1):
            rdma = pltpu.make_async_remote_copy(
                src_ref=comm_ref.at[s],
                dst_ref=comm_ref.at[s + 1],
                send_sem=send_sems.at[s],
                recv_sem=recv_sems.at[s],
                device_id=(right,),
                device_id_type=pl.DeviceIdType.MESH,
            )
            rdma.start()
            if s + 2 < N_DEV:
                loads.append(start_load(ridx(my - s - 2), s % 2))
            for c in loads[s + 1]:
                c.wait()
            compute_batch(ridx(my - s - 1), (s + 1) % 2, part_ref.at[s])
            rdma.wait()
            acc = comm_ref[s + 1].astype(jnp.float32) + part_ref[s].astype(
                jnp.float32
            )
            comm_ref[s + 1] = acc.astype(jnp.bfloat16)

        g = lax.rem(my + 1, N_DEV)
        out_ref[pl.ds(g * mc, mc), :] = comm_ref[N_DEV - 1].astype(jnp.float32)

        ag_rdmas = []
        for j in range(1, N_DEV):
            tgt = lax.rem(my + j, N_DEV)
            m_slot = N_DEV - 1 - j
            rdma = pltpu.make_async_remote_copy(
                src_ref=comm_ref.at[N_DEV - 1],
                dst_ref=comm_ref.at[N_DEV + m_slot],
                send_sem=ag_send_sems.at[j - 1],
                recv_sem=ag_recv_sems.at[m_slot],
                device_id=(tgt,),
                device_id_type=pl.DeviceIdType.MESH,
            )
            rdma.start()
            ag_rdmas.append(rdma)

        for m_slot in (0, 2, 1):
            recv = pltpu.make_async_remote_copy(
                src_ref=comm_ref.at[N_DEV - 1],
                dst_ref=comm_ref.at[N_DEV + m_slot],
                send_sem=ag_send_sems.at[0],
                recv_sem=ag_recv_sems.at[m_slot],
                device_id=(my,),
                device_id_type=pl.DeviceIdType.MESH,
            )
            recv.wait_recv()
            origin = lax.rem(my + m_slot + 2, N_DEV)
            out_ref[pl.ds(origin * mc, mc), :] = comm_ref[
                N_DEV + m_slot
            ].astype(jnp.float32)

        for rdma in ag_rdmas:
            rdma.wait_send()

    return pl.pallas_call(
        body,
        out_shape=jax.ShapeDtypeStruct((m, d), jnp.float32),
        in_specs=[
            pl.BlockSpec(memory_space=pltpu.VMEM),
            pl.BlockSpec(memory_space=pltpu.VMEM),
            pl.BlockSpec(memory_space=pltpu.VMEM),
            pl.BlockSpec(memory_space=pltpu.MemorySpace.HBM),
            pl.BlockSpec(memory_space=pltpu.MemorySpace.HBM),
        ],
        out_specs=pl.BlockSpec(memory_space=pltpu.VMEM),
        scratch_shapes=[
            pltpu.VMEM((mc, h_loc * dh), jnp.bfloat16),
            pltpu.VMEM((N_DEV - 1, mc, d), jnp.bfloat16),
            pltpu.VMEM((2 * N_DEV - 1, mc, d), jnp.bfloat16),
            pltpu.VMEM((2, h_loc, skv, dh), jnp.float32),
            pltpu.VMEM((2, h_loc, skv, dh), jnp.float32),
            pltpu.SemaphoreType.DMA((2,)),
            pltpu.SemaphoreType.DMA((N_DEV - 1,)),
            pltpu.SemaphoreType.DMA((N_DEV - 1,)),
            pltpu.SemaphoreType.DMA((N_DEV - 1,)),
            pltpu.SemaphoreType.DMA((N_DEV - 1,)),
        ],
        compiler_params=pltpu.CompilerParams(collective_id=0),
    )(x2, wq, wo, k_ext, v_ext)


def kernel(x, Wq, Wo, K_ext, V_ext):
    b, sq, d = x.shape

    x2 = x.reshape(b * sq, d).astype(jnp.bfloat16)
    wq = (Wq * SCALE).astype(jnp.bfloat16)
    wo = Wo.astype(jnp.bfloat16)

    out = _fused(x2, wq, wo, K_ext, V_ext)
    return out.reshape(b, sq, d)


# device time: 52552 ns/iter; 4.6872x vs baseline; 1.0007x over previous
import jax
import jax.numpy as jnp
from jax import lax
from jax.experimental import pallas as pl
from jax.experimental.pallas import tpu as pltpu

N_DEV = 4
SCALE = 0.08838834764831843


def _fused(x2, wq, wo, k_ext, v_ext):
    m, d = x2.shape
    b_sz, skv, h_tot, dh = k_ext.shape
    h_loc = wq.shape[1] // dh
    mc = m // N_DEV

    def body(x_ref, wq_ref, wo_ref, khbm_ref, vhbm_ref, out_ref,
             o_ref, part_ref, comm_ref, kbuf, vbuf, load_sems,
             send_sems, recv_sems, ag_send_sems, ag_recv_sems):
        my = lax.axis_index("i")
        left = lax.rem(my + (N_DEV - 1), N_DEV)
        right = lax.rem(my + 1, N_DEV)
        base = my * h_loc

        def ridx(idx):
            return lax.rem(idx + 2 * N_DEV, N_DEV)

        def start_load(bidx, slot):
            copies = []
            for h in range(h_loc):
                copies.append(pltpu.make_async_copy(
                    khbm_ref.at[bidx, :, base + h, :],
                    kbuf.at[slot, h], load_sems.at[slot]))
                copies.append(pltpu.make_async_copy(
                    vhbm_ref.at[bidx, :, base + h, :],
                    vbuf.at[slot, h], load_sems.at[slot]))
            for c in copies:
                c.start()
            return copies

        loads = [start_load(my, 0), start_load(ridx(my - 1), 1)]

        barrier_sem = pltpu.get_barrier_semaphore()
        for nbr in (left, right):
            pl.semaphore_signal(
                barrier_sem, inc=1,
                device_id=(nbr,), device_id_type=pl.DeviceIdType.MESH,
            )
        pl.semaphore_wait(barrier_sem, 2)

        def compute_batch(bidx, slot, dst):
            qb = jnp.dot(
                x_ref[pl.ds(bidx * mc, mc), :], wq_ref[...],
                preferred_element_type=jnp.float32,
            ).astype(jnp.bfloat16)
            for h in range(h_loc):
                s = lax.dot_general(
                    qb[:, h * dh:(h + 1) * dh],
                    kbuf[slot, h].astype(jnp.bfloat16),
                    (((1,), (1,)), ((), ())),
                    preferred_element_type=jnp.float32,
                )
                e = jnp.exp(s.astype(jnp.bfloat16))
                o = jnp.dot(
                    e,
                    vbuf[slot, h].astype(jnp.bfloat16),
                    preferred_element_type=jnp.float32,
                ) / jnp.sum(e, axis=1, keepdims=True, dtype=jnp.float32)
                o_ref[:, h * dh:(h + 1) * dh] = o.astype(jnp.bfloat16)
            dst[...] = jnp.dot(
                o_ref[...], wo_ref[...], preferred_element_type=jnp.float32
            ).astype(jnp.bfloat16)

        for c in loads[0]:
            c.wait()
        compute_batch(my, 0, comm_ref.at[0])
        for s in range(N_DEV - 1):
            rdma = pltpu.make_async_remote_copy(
                src_ref=comm_ref.at[s],
                dst_ref=comm_ref.at[s + 1],
                send_sem=send_sems.at[s],
                recv_sem=recv_sems.at[s],
                device_id=(right,),
                device_id_type=pl.DeviceIdType.MESH,
            )
            rdma.start()
            if s + 2 < N_DEV:
                loads.append(start_load(ridx(my - s - 2), s % 2))
            for c in loads[s + 1]:
                c.wait()
            compute_batch(ridx(my - s - 1), (s + 1) % 2, part_ref.at[s])
            rdma.wait()
            acc = comm_ref[s + 1].astype(jnp.float32) + part_ref[s].astype(
                jnp.float32
            )
            comm_ref[s + 1] = acc.astype(jnp.bfloat16)

        g = lax.rem(my + 1, N_DEV)
        out_ref[pl.ds(g * mc, mc), :] = comm_ref[N_DEV - 1].astype(jnp.float32)

        ag_rdmas = []
        for j in range(1, N_DEV):
            tgt = lax.rem(my + j, N_DEV)
            m_slot = N_DEV - 1 - j
            rdma = pltpu.make_async_remote_copy(
                src_ref=comm_ref.at[N_DEV - 1],
                dst_ref=comm_ref.at[N_DEV + m_slot],
                send_sem=ag_send_sems.at[j - 1],
                recv_sem=ag_recv_sems.at[m_slot],
                device_id=(tgt,),
                device_id_type=pl.DeviceIdType.MESH,
            )
            rdma.start()
            ag_rdmas.append(rdma)

        for m_slot in (0, 2, 1):
            recv = pltpu.make_async_remote_copy(
                src_ref=comm_ref.at[N_DEV - 1],
                dst_ref=comm_ref.at[N_DEV + m_slot],
                send_sem=ag_send_sems.at[0],
                recv_sem=ag_recv_sems.at[m_slot],
                device_id=(my,),
                device_id_type=pl.DeviceIdType.MESH,
            )
            recv.wait_recv()
            origin = lax.rem(my + m_slot + 2, N_DEV)
            out_ref[pl.ds(origin * mc, mc), :] = comm_ref[
                N_DEV + m_slot
            ].astype(jnp.float32)

        for rdma in ag_rdmas:
            rdma.wait_send()

    return pl.pallas_call(
        body,
        out_shape=jax.ShapeDtypeStruct((m, d), jnp.float32),
        in_specs=[
            pl.BlockSpec(memory_space=pltpu.VMEM),
            pl.BlockSpec(memory_space=pltpu.VMEM),
            pl.BlockSpec(memory_space=pltpu.VMEM),
            pl.BlockSpec(memory_space=pltpu.MemorySpace.HBM),
            pl.BlockSpec(memory_space=pltpu.MemorySpace.HBM),
        ],
        out_specs=pl.BlockSpec(memory_space=pltpu.VMEM),
        scratch_shapes=[
            pltpu.VMEM((mc, h_loc * dh), jnp.bfloat16),
            pltpu.VMEM((N_DEV - 1, mc, d), jnp.bfloat16),
            pltpu.VMEM((2 * N_DEV - 1, mc, d), jnp.bfloat16),
            pltpu.VMEM((2, h_loc, skv, dh), jnp.float32),
            pltpu.VMEM((2, h_loc, skv, dh), jnp.float32),
            pltpu.SemaphoreType.DMA((2,)),
            pltpu.SemaphoreType.DMA((N_DEV - 1,)),
            pltpu.SemaphoreType.DMA((N_DEV - 1,)),
            pltpu.SemaphoreType.DMA((N_DEV - 1,)),
            pltpu.SemaphoreType.DMA((N_DEV - 1,)),
        ],
        compiler_params=pltpu.CompilerParams(collective_id=0),
    )(x2, wq, wo, k_ext, v_ext)


def kernel(x, Wq, Wo, K_ext, V_ext):
    b, sq, d = x.shape

    x2 = x.reshape(b * sq, d).astype(jnp.bfloat16)
    wq = (Wq * SCALE).astype(jnp.bfloat16)
    wo = Wo.astype(jnp.bfloat16)

    out = _fused(x2, wq, wo, K_ext, V_ext)
    return out.reshape(b, sq, d)
